# Initial kernel scaffold; baseline (speedup 1.0000x reference)
import jax
import jax.numpy as jnp
from jax.experimental import pallas as pl


def kernel(msa, pair, xyz, seq1hot, idx, top_k, ln_msa_g, ln_msa_b, ln_pair_g, ln_pair_b, Wq, bq, Wk, bk, Wx, bx, We, be, ln_node_g, ln_node_b, ln_edge_g, ln_edge_b, W1, b1, W0, b0, Wr, br, Wself, bself):
    raise NotImplementedError("write your pallas kernel here")


# speedup vs baseline: 18.1570x; 18.1570x over previous
import math

import jax
import jax.numpy as jnp
from jax.experimental import pallas as pl
from jax.experimental.pallas import tpu as pltpu

_EPS_LN = 1e-5
_TILE = 8


def _prologue_body(msa_ref, seq_ref, cac_ref, car_ref, idc_ref, idr_ref,
                   kv_ref, gm_ref, bm_ref, wq_ref, bq_ref, wk_ref, bk_ref,
                   wxm_ref, wxs_ref, bx_ref, gn_ref, bn_ref, w1n_ref, b1_ref,
                   maskT_ref, dist_ref, node_ref, nw1_ref):
    NL, D = msa_ref.shape
    Lh = node_ref.shape[0]
    N = NL // Lh

    msa = msa_ref[...]
    mu = jnp.mean(msa, axis=-1, keepdims=True)
    var = jnp.mean((msa - mu) ** 2, axis=-1, keepdims=True)
    msa_n = (msa - mu) * jax.lax.rsqrt(var + _EPS_LN) * gm_ref[...] + bm_ref[...]

    tar = msa_n[0:Lh, :]
    q = (jnp.dot(tar, wq_ref[...], preferred_element_type=jnp.float32)
         + bq_ref[...]) * (1.0 / math.sqrt(D))
    kk = jnp.dot(msa_n, wk_ref[...], preferred_element_type=jnp.float32) + bk_ref[...]
    logits = jnp.concatenate(
        [jnp.sum(q * kk[n * Lh:(n + 1) * Lh, :], axis=-1, keepdims=True)
         for n in range(N)], axis=1)
    mx = jnp.max(logits, axis=-1, keepdims=True)
    ex = jnp.exp(logits - mx)
    attn = ex / jnp.sum(ex, axis=-1, keepdims=True)
    msa_w = attn[:, 0:1] * msa_n[0:Lh, :]
    for n in range(1, N):
        msa_w = msa_w + attn[:, n:n + 1] * msa_n[n * Lh:(n + 1) * Lh, :]

    pre = (jnp.dot(msa_w, wxm_ref[...], preferred_element_type=jnp.float32)
           + jnp.dot(seq_ref[...], wxs_ref[...], preferred_element_type=jnp.float32)
           + bx_ref[...])
    mu2 = jnp.mean(pre, axis=-1, keepdims=True)
    var2 = jnp.mean((pre - mu2) ** 2, axis=-1, keepdims=True)
    node = (pre - mu2) * jax.lax.rsqrt(var2 + _EPS_LN) * gn_ref[...] + bn_ref[...]
    node_ref[...] = node
    nw1_ref[...] = jnp.dot(node, w1n_ref[...],
                           preferred_element_type=jnp.float32) + b1_ref[...]

    d2 = jnp.zeros((Lh, Lh), jnp.float32)
    for x in range(3):
        dx = cac_ref[:, x:x + 1] - car_ref[x:x + 1, :]
        d2 = d2 + dx * dx
    dist = jnp.sqrt(d2 + 1e-12)
    dist_ref[...] = dist

    ii = jax.lax.broadcasted_iota(jnp.int32, (Lh, Lh), 0)
    jj = jax.lax.broadcasted_iota(jnp.int32, (Lh, Lh), 1)
    eye = ii == jj
    dm = jnp.where(eye, dist + 999.9, dist)

    kv = kv_ref[...]
    lo0 = jnp.zeros((Lh, 1), jnp.float32)
    hi0 = jnp.full((Lh, 1), 2000.0, jnp.float32)

    def body(_, carry):
        lo, hi = carry
        mid = (lo + hi) * 0.5
        cnt = jnp.sum((dm <= mid).astype(jnp.float32), axis=-1, keepdims=True)
        take = cnt >= kv
        return jnp.where(take, lo, mid), jnp.where(take, mid, hi)

    _, thr_col = jax.lax.fori_loop(0, 48, body, (lo0, hi0))
    eyef = eye.astype(jnp.float32)
    thr_row = jnp.sum(eyef * thr_col, axis=0, keepdims=True)

    sep = jnp.abs(idc_ref[...] - idr_ref[...])
    sep = jnp.where(eye, sep + 999.9, sep)
    m_t = jnp.logical_or(dm <= thr_row, sep < 9.0)
    maskT_ref[...] = m_t.astype(jnp.float32)


def _main_body(xyz_s, br_s, pair_ref, mT_ref, dT_ref, nw1_ref,
               gp_ref, bp_ref, we_ref, be_ref, ge_ref, bee_ref,
               w1e_ref, w1d_ref, w0_ref, b0_ref, wr_ref, brr_ref,
               wself_ref, bself_ref, node_ref, cac_ref,
               xyz_out, st_out, h_acc, deg_acc, a1_acc):
    i = pl.program_id(0)
    nsteps = pl.num_programs(0)
    Lh = mT_ref.shape[1]
    T = mT_ref.shape[2]

    @pl.when(i == 0)
    def _init():
        h_acc[...] = jnp.zeros_like(h_acc)
        deg_acc[...] = jnp.zeros_like(deg_acc)
        a1_acc[...] = jnp.zeros_like(a1_acc)

    p = pair_ref[...]
    mu = jnp.mean(p, axis=-1, keepdims=True)
    var = jnp.mean((p - mu) ** 2, axis=-1, keepdims=True)
    pn = (p - mu) * jax.lax.rsqrt(var + _EPS_LN) * gp_ref[...] + bp_ref[...]
    pe = jnp.dot(pn, we_ref[...], preferred_element_type=jnp.float32) + be_ref[...]
    mu2 = jnp.mean(pe, axis=-1, keepdims=True)
    var2 = jnp.mean((pe - mu2) ** 2, axis=-1, keepdims=True)
    pe = (pe - mu2) * jax.lax.rsqrt(var2 + _EPS_LN) * ge_ref[...] + bee_ref[...]
    base = jnp.dot(pe, w1e_ref[...], preferred_element_type=jnp.float32)

    w1d = w1d_ref[...]
    wr = wr_ref[...]

    h_new = h_acc[...]
    deg_new = deg_acc[...]
    a1_new = a1_acc[...]
    for t in range(T):
        ig = i * T + t
        distc = dT_ref[0, :, t:t + 1]
        mcol = mT_ref[0, :, t:t + 1]
        hid = jnp.maximum(
            base[t * Lh:(t + 1) * Lh, :] + distc * w1d + nw1_ref[t:t + 1, :], 0.0)
        hm = mcol * hid
        h_new = h_new + hm
        deg_new = deg_new + mcol
        rw = jnp.dot(hm, wr, preferred_element_type=jnp.float32)

        ca = [xyz_s[ig * 9 + 3 + x] for x in range(3)]
        v = [[xyz_s[ig * 9 + 3 * k + x] - ca[x] for x in range(3)]
             for k in range(3)]
        cols = []
        for c in range(3):
            for x in range(3):
                col = rw[:, c:c + 1] * (-ca[x])
                cbr = -br_s[c] * ca[x]
                for k in range(3):
                    col = col + rw[:, 3 + 3 * c + k:4 + 3 * c + k] * v[k][x]
                    cbr = cbr + br_s[3 + 3 * c + k] * v[k][x]
                cols.append(col + mcol * cbr)
        a1_new = a1_new + jnp.concatenate(cols, axis=1)
    h_acc[...] = h_new
    deg_acc[...] = deg_new
    a1_acc[...] = a1_new

    @pl.when(i == nsteps - 1)
    def _finish():
        hf = h_acc[...]
        degf = deg_acc[...]
        rwsum = jnp.dot(hf, wr, preferred_element_type=jnp.float32) + degf * brr_ref[...]
        cav = cac_ref[...]
        add = jnp.concatenate(
            [cav[:, x:x + 1] * rwsum[:, c:c + 1]
             for c in range(3) for x in range(3)], axis=1)
        off = a1_acc[...] + add
        agg0 = (jnp.dot(hf, w0_ref[...], preferred_element_type=jnp.float32)
                + degf * b0_ref[...])
        st_out[...] = (agg0
                       + jnp.dot(node_ref[...], wself_ref[...],
                                 preferred_element_type=jnp.float32)
                       + bself_ref[...])
        ca_new = cav + off[:, 3:6]
        xyz_out[...] = jnp.concatenate(
            [ca_new + off[:, 0:3], ca_new, ca_new + off[:, 6:9]], axis=1)


def kernel(msa, pair, xyz, seq1hot, idx, top_k, ln_msa_g, ln_msa_b, ln_pair_g,
           ln_pair_b, Wq, bq, Wk, bk, Wx, bx, We, be, ln_node_g, ln_node_b,
           ln_edge_g, ln_edge_b, W1, b1, W0, b0, Wr, br, Wself, bself):
    B, N, L, D = msa.shape
    DP = pair.shape[-1]
    L0 = Wx.shape[1]
    HID = W1.shape[1]
    L0O = W0.shape[1]

    msa2d = msa.reshape(B * N * L, D)
    seq2d = seq1hot.reshape(B * L, seq1hot.shape[-1])
    xyzf = xyz.reshape(B * L, 9)
    ca_col = xyzf[:, 3:6]
    ca_row = ca_col.T
    idx_col = idx.reshape(B * L, 1).astype(jnp.float32)
    idx_row = idx_col.T
    kval = jnp.minimum(jnp.asarray(top_k, jnp.float32), float(L)).reshape(1, 1)
    pair2 = pair.reshape(B * L * L, DP)

    row = lambda a: a.reshape(1, -1)
    f32 = jnp.float32

    maskT, distm, node, nw1 = pl.pallas_call(
        _prologue_body,
        out_shape=[
            jax.ShapeDtypeStruct((L, L), f32),
            jax.ShapeDtypeStruct((L, L), f32),
            jax.ShapeDtypeStruct((L, L0), f32),
            jax.ShapeDtypeStruct((L, HID), f32),
        ],
    )(msa2d, seq2d, ca_col, ca_row, idx_col, idx_row, kval,
      row(ln_msa_g), row(ln_msa_b), Wq, row(bq), Wk, row(bk),
      Wx[:D, :], Wx[D:, :], row(bx), row(ln_node_g), row(ln_node_b),
      W1[:L0, :], row(b1))

    T = _TILE
    nsteps = L // T
    full = lambda shape: pl.BlockSpec(shape, lambda i, *_: (0, 0))
    grid_spec = pltpu.PrefetchScalarGridSpec(
        num_scalar_prefetch=2,
        grid=(nsteps,),
        in_specs=[
            pl.BlockSpec((T * L, DP), lambda i, *_: (i, 0)),
            pl.BlockSpec((1, L, T), lambda i, *_: (i, 0, 0)),
            pl.BlockSpec((1, L, T), lambda i, *_: (i, 0, 0)),
            pl.BlockSpec((T, HID), lambda i, *_: (i, 0)),
            full((1, DP)), full((1, DP)),
            full((DP, We.shape[1])), full((1, We.shape[1])),
            full((1, We.shape[1])), full((1, We.shape[1])),
            full((We.shape[1], HID)), full((1, HID)),
            full((HID, L0O)), full((1, L0O)),
            full((HID, 12)), full((1, 12)),
            full((L0, L0O)), full((1, L0O)),
            full((L, L0)),
            full((L, 3)),
        ],
        out_specs=[
            pl.BlockSpec((L, 9), lambda i, *_: (0, 0)),
            pl.BlockSpec((L, L0O), lambda i, *_: (0, 0)),
        ],
        scratch_shapes=[
            pltpu.VMEM((L, HID), f32),
            pltpu.VMEM((L, 1), f32),
            pltpu.VMEM((L, 9), f32),
        ],
    )
    xyz_flat, state = pl.pallas_call(
        _main_body,
        grid_spec=grid_spec,
        out_shape=[
            jax.ShapeDtypeStruct((L, 9), f32),
            jax.ShapeDtypeStruct((L, L0O), f32),
        ],
    )(xyzf.reshape(-1), br, pair2,
      maskT.reshape(L, nsteps, T).transpose(1, 0, 2),
      distm.reshape(L, nsteps, T).transpose(1, 0, 2), nw1,
      row(ln_pair_g), row(ln_pair_b), We, row(be), row(ln_edge_g),
      row(ln_edge_b), W1[L0:L0 + We.shape[1], :], W1[L0 + We.shape[1]:, :],
      W0, row(b0), Wr, row(br), Wself, row(bself), node, ca_col)

    xyz_new = xyz_flat.reshape(B, L, 3, 3)
    return xyz_new, state.reshape(B, L, L0O)


# speedup vs baseline: 101.2450x; 5.5761x over previous
import math

import jax
import jax.numpy as jnp
from jax.experimental import pallas as pl
from jax.experimental.pallas import tpu as pltpu

_EPS_LN = 1e-5
_TILE = 8


def _prologue_body(msa_ref, seq_ref, cac_ref, car_ref, idc_ref, idr_ref,
                   kv_ref, gm_ref, bm_ref, wq_ref, bq_ref, wk_ref, bk_ref,
                   wxm_ref, wxs_ref, bx_ref, gn_ref, bn_ref, w1n_ref, b1_ref,
                   maskT_ref, dist_ref, node_ref, nw1_ref):
    NL, D = msa_ref.shape
    Lh = node_ref.shape[0]
    N = NL // Lh

    msa = msa_ref[...]
    mu = jnp.mean(msa, axis=-1, keepdims=True)
    var = jnp.mean((msa - mu) ** 2, axis=-1, keepdims=True)
    msa_n = (msa - mu) * jax.lax.rsqrt(var + _EPS_LN) * gm_ref[...] + bm_ref[...]

    tar = msa_n[0:Lh, :]
    q = (jnp.dot(tar, wq_ref[...], preferred_element_type=jnp.float32)
         + bq_ref[...]) * (1.0 / math.sqrt(D))
    kk = jnp.dot(msa_n, wk_ref[...], preferred_element_type=jnp.float32) + bk_ref[...]
    logits = jnp.concatenate(
        [jnp.sum(q * kk[n * Lh:(n + 1) * Lh, :], axis=-1, keepdims=True)
         for n in range(N)], axis=1)
    mx = jnp.max(logits, axis=-1, keepdims=True)
    ex = jnp.exp(logits - mx)
    attn = ex / jnp.sum(ex, axis=-1, keepdims=True)
    msa_w = attn[:, 0:1] * msa_n[0:Lh, :]
    for n in range(1, N):
        msa_w = msa_w + attn[:, n:n + 1] * msa_n[n * Lh:(n + 1) * Lh, :]

    pre = (jnp.dot(msa_w, wxm_ref[...], preferred_element_type=jnp.float32)
           + jnp.dot(seq_ref[...], wxs_ref[...], preferred_element_type=jnp.float32)
           + bx_ref[...])
    mu2 = jnp.mean(pre, axis=-1, keepdims=True)
    var2 = jnp.mean((pre - mu2) ** 2, axis=-1, keepdims=True)
    node = (pre - mu2) * jax.lax.rsqrt(var2 + _EPS_LN) * gn_ref[...] + bn_ref[...]
    node_ref[...] = node
    nw1_ref[...] = jnp.dot(node, w1n_ref[...],
                           preferred_element_type=jnp.float32) + b1_ref[...]

    d2 = jnp.zeros((Lh, Lh), jnp.float32)
    for x in range(3):
        dx = cac_ref[:, x:x + 1] - car_ref[x:x + 1, :]
        d2 = d2 + dx * dx
    dist = jnp.sqrt(d2 + 1e-12)
    dist_ref[...] = dist

    ii = jax.lax.broadcasted_iota(jnp.int32, (Lh, Lh), 0)
    jj = jax.lax.broadcasted_iota(jnp.int32, (Lh, Lh), 1)
    eye = ii == jj
    dm = jnp.where(eye, dist + 999.9, dist)

    kv = kv_ref[...]
    lo0 = jnp.zeros((Lh, 1), jnp.float32)
    hi0 = jnp.full((Lh, 1), 2000.0, jnp.float32)

    def body(_, carry):
        lo, hi = carry
        mid = (lo + hi) * 0.5
        cnt = jnp.sum((dm <= mid).astype(jnp.float32), axis=-1, keepdims=True)
        take = cnt >= kv
        return jnp.where(take, lo, mid), jnp.where(take, mid, hi)

    _, thr_col = jax.lax.fori_loop(0, 48, body, (lo0, hi0))
    eyef = eye.astype(jnp.float32)
    thr_row = jnp.sum(eyef * thr_col, axis=0, keepdims=True)

    sep = jnp.abs(idc_ref[...] - idr_ref[...])
    sep = jnp.where(eye, sep + 999.9, sep)
    m_t = jnp.logical_or(dm <= thr_row, sep < 9.0)
    maskT_ref[...] = m_t.astype(jnp.float32)


def _main_body(xyz_s, pair_ref, mT_ref, dT_ref, nw1_ref,
               gp_ref, bp_ref, we_ref, be_ref, ge_ref, bee_ref,
               w1e_ref, w1d_ref, w0_ref, b0_ref, wr_ref, brr_ref,
               wself_ref, bself_ref, node_ref, cac_ref,
               xyz_out, st_out, h_acc, deg_acc, a1_acc):
    i = pl.program_id(0)
    nsteps = pl.num_programs(0)
    Lh = mT_ref.shape[1]
    T = mT_ref.shape[2]
    A1W = a1_acc.shape[1]

    @pl.when(i == 0)
    def _init():
        h_acc[...] = jnp.zeros_like(h_acc)
        deg_acc[...] = jnp.zeros_like(deg_acc)
        a1_acc[...] = jnp.zeros_like(a1_acc)

    p = pair_ref[...]
    mu = jnp.mean(p, axis=-1, keepdims=True)
    var = jnp.mean((p - mu) ** 2, axis=-1, keepdims=True)
    pn = (p - mu) * jax.lax.rsqrt(var + _EPS_LN) * gp_ref[...] + bp_ref[...]
    pe = jnp.dot(pn, we_ref[...], preferred_element_type=jnp.float32) + be_ref[...]
    mu2 = jnp.mean(pe, axis=-1, keepdims=True)
    var2 = jnp.mean((pe - mu2) ** 2, axis=-1, keepdims=True)
    pe = (pe - mu2) * jax.lax.rsqrt(var2 + _EPS_LN) * ge_ref[...] + bee_ref[...]
    base = jnp.dot(pe, w1e_ref[...], preferred_element_type=jnp.float32)

    w1d = w1d_ref[...]
    wr = wr_ref[...]
    brr = brr_ref[...]

    lane = jax.lax.broadcasted_iota(jnp.int32, (1, A1W), 1)
    px = [((lane % 3 == x) & (lane < 9)).astype(jnp.float32) for x in range(3)]
    zpad = jnp.zeros((wr.shape[0], A1W - 9), jnp.float32)
    wra = jnp.concatenate(
        [wr[:, c:c + 1] for c in range(3) for _ in range(3)] + [zpad], axis=1)
    wrb = [jnp.concatenate(
        [wr[:, 3 + 3 * c + k:4 + 3 * c + k] for c in range(3) for _ in range(3)]
        + [zpad], axis=1) for k in range(3)]
    zrow = jnp.zeros((1, A1W - 9), jnp.float32)
    bra = jnp.concatenate(
        [brr[:, c:c + 1] for c in range(3) for _ in range(3)] + [zrow], axis=1)
    brb = [jnp.concatenate(
        [brr[:, 3 + 3 * c + k:4 + 3 * c + k] for c in range(3) for _ in range(3)]
        + [zrow], axis=1) for k in range(3)]

    h_new = h_acc[...]
    deg_new = deg_acc[...]
    a1_new = a1_acc[...]
    for t in range(T):
        ig = i * T + t
        distc = dT_ref[0, :, t:t + 1]
        mcol = mT_ref[0, :, t:t + 1]
        hid = jnp.maximum(
            base[t * Lh:(t + 1) * Lh, :] + distc * w1d + nw1_ref[t:t + 1, :], 0.0)
        hm = mcol * hid
        h_new = h_new + hm
        deg_new = deg_new + mcol

        ca = [xyz_s[ig * 9 + 3 + x] for x in range(3)]
        v = [[xyz_s[ig * 9 + 3 * k + x] - ca[x] for x in range(3)]
             for k in range(3)]
        ca_row = ca[0] * px[0] + ca[1] * px[1] + ca[2] * px[2]
        v_row = [v[k][0] * px[0] + v[k][1] * px[1] + v[k][2] * px[2]
                 for k in range(3)]
        mt = -wra * ca_row + wrb[0] * v_row[0] + wrb[1] * v_row[1] + wrb[2] * v_row[2]
        cbr = -bra * ca_row + brb[0] * v_row[0] + brb[1] * v_row[1] + brb[2] * v_row[2]
        a1_new = (a1_new + jnp.dot(hm, mt, preferred_element_type=jnp.float32)
                  + mcol * cbr)
    h_acc[...] = h_new
    deg_acc[...] = deg_new
    a1_acc[...] = a1_new

    @pl.when(i == nsteps - 1)
    def _finish():
        hf = h_acc[...]
        degf = deg_acc[...]
        rwsum = jnp.dot(hf, wr, preferred_element_type=jnp.float32) + degf * brr_ref[...]
        cav = cac_ref[...]
        add = jnp.concatenate(
            [cav[:, x:x + 1] * rwsum[:, c:c + 1]
             for c in range(3) for x in range(3)], axis=1)
        off = a1_acc[:, 0:9] + add
        agg0 = (jnp.dot(hf, w0_ref[...], preferred_element_type=jnp.float32)
                + degf * b0_ref[...])
        st_out[...] = (agg0
                       + jnp.dot(node_ref[...], wself_ref[...],
                                 preferred_element_type=jnp.float32)
                       + bself_ref[...])
        ca_new = cav + off[:, 3:6]
        xyz_out[...] = jnp.concatenate(
            [ca_new + off[:, 0:3], ca_new, ca_new + off[:, 6:9]], axis=1)


def kernel(msa, pair, xyz, seq1hot, idx, top_k, ln_msa_g, ln_msa_b, ln_pair_g,
           ln_pair_b, Wq, bq, Wk, bk, Wx, bx, We, be, ln_node_g, ln_node_b,
           ln_edge_g, ln_edge_b, W1, b1, W0, b0, Wr, br, Wself, bself):
    B, N, L, D = msa.shape
    DP = pair.shape[-1]
    L0 = Wx.shape[1]
    HID = W1.shape[1]
    L0O = W0.shape[1]

    msa2d = msa.reshape(B * N * L, D)
    seq2d = seq1hot.reshape(B * L, seq1hot.shape[-1])
    xyzf = xyz.reshape(B * L, 9)
    ca_col = xyzf[:, 3:6]
    ca_row = ca_col.T
    idx_col = idx.reshape(B * L, 1).astype(jnp.float32)
    idx_row = idx_col.T
    kval = jnp.minimum(jnp.asarray(top_k, jnp.float32), float(L)).reshape(1, 1)
    pair2 = pair.reshape(B * L * L, DP)

    row = lambda a: a.reshape(1, -1)
    f32 = jnp.float32

    maskT, distm, node, nw1 = pl.pallas_call(
        _prologue_body,
        out_shape=[
            jax.ShapeDtypeStruct((L, L), f32),
            jax.ShapeDtypeStruct((L, L), f32),
            jax.ShapeDtypeStruct((L, L0), f32),
            jax.ShapeDtypeStruct((L, HID), f32),
        ],
    )(msa2d, seq2d, ca_col, ca_row, idx_col, idx_row, kval,
      row(ln_msa_g), row(ln_msa_b), Wq, row(bq), Wk, row(bk),
      Wx[:D, :], Wx[D:, :], row(bx), row(ln_node_g), row(ln_node_b),
      W1[:L0, :], row(b1))

    T = _TILE
    nsteps = L // T
    full = lambda shape: pl.BlockSpec(shape, lambda i, *_: (0, 0))
    grid_spec = pltpu.PrefetchScalarGridSpec(
        num_scalar_prefetch=1,
        grid=(nsteps,),
        in_specs=[
            pl.BlockSpec((T * L, DP), lambda i, *_: (i, 0)),
            pl.BlockSpec((1, L, T), lambda i, *_: (i, 0, 0)),
            pl.BlockSpec((1, L, T), lambda i, *_: (i, 0, 0)),
            pl.BlockSpec((T, HID), lambda i, *_: (i, 0)),
            full((1, DP)), full((1, DP)),
            full((DP, We.shape[1])), full((1, We.shape[1])),
            full((1, We.shape[1])), full((1, We.shape[1])),
            full((We.shape[1], HID)), full((1, HID)),
            full((HID, L0O)), full((1, L0O)),
            full((HID, 12)), full((1, 12)),
            full((L0, L0O)), full((1, L0O)),
            full((L, L0)),
            full((L, 3)),
        ],
        out_specs=[
            pl.BlockSpec((L, 9), lambda i, *_: (0, 0)),
            pl.BlockSpec((L, L0O), lambda i, *_: (0, 0)),
        ],
        scratch_shapes=[
            pltpu.VMEM((L, HID), f32),
            pltpu.VMEM((L, 1), f32),
            pltpu.VMEM((L, 16), f32),
        ],
    )
    xyz_flat, state = pl.pallas_call(
        _main_body,
        grid_spec=grid_spec,
        out_shape=[
            jax.ShapeDtypeStruct((L, 9), f32),
            jax.ShapeDtypeStruct((L, L0O), f32),
        ],
    )(xyzf.reshape(-1), pair2,
      maskT.reshape(L, nsteps, T).transpose(1, 0, 2),
      distm.reshape(L, nsteps, T).transpose(1, 0, 2), nw1,
      row(ln_pair_g), row(ln_pair_b), We, row(be), row(ln_edge_g),
      row(ln_edge_b), W1[L0:L0 + We.shape[1], :], W1[L0 + We.shape[1]:, :],
      W0, row(b0), Wr, row(br), Wself, row(bself), node, ca_col)

    xyz_new = xyz_flat.reshape(B, L, 3, 3)
    return xyz_new, state.reshape(B, L, L0O)
